# baseline (speedup 1.0000x reference)
import functools

import jax
import jax.numpy as jnp
from jax import lax
from jax.experimental import pallas as pl
from jax.experimental.pallas import tpu as pltpu
from jax.experimental.pallas import tpu_sc as plsc

_NC = 2
_NS = 16
_NW = _NC * _NS
_L = 16

_BLK = 32768
_V0 = 18 * _BLK
_V1 = 999424
_CH = 512


def _matvec_body(tab_ref, w_ref, b_ref, t_ref):
    x = tab_ref[...]
    w = w_ref[...]
    t_ref[...] = jnp.sum(x * w, axis=0, keepdims=True) + b_ref[0]


def _tc_matvec(tableT, WT, b, n_cols, blk):
    D = tableT.shape[0]
    grid = n_cols // blk
    return pl.pallas_call(
        _matvec_body,
        grid=(grid,),
        in_specs=[
            pl.BlockSpec((D, blk), lambda i: (0, i)),
            pl.BlockSpec((D, 1), lambda i: (0, 0)),
            pl.BlockSpec((1,), lambda i: (0,)),
        ],
        out_specs=pl.BlockSpec((1, blk), lambda i: (0, i)),
        out_shape=jax.ShapeDtypeStruct((1, n_cols), jnp.float32),
    )(tableT, WT, b)


def _sc_matvec(tableT, w_bcast, b_bcast, v0, n_cols):
    D = tableT.shape[0]
    per_w = n_cols // _NW
    n_chunks = per_w // _CH
    n_grp = _CH // _L
    mesh = plsc.VectorSubcoreMesh(core_axis_name="c", subcore_axis_name="s")

    @functools.partial(
        pl.kernel,
        out_type=jax.ShapeDtypeStruct((n_cols,), jnp.float32),
        mesh=mesh,
        scratch_types=[
            pltpu.VMEM((D, _CH), jnp.float32),
            pltpu.VMEM((D, _CH), jnp.float32),
            pltpu.VMEM((D, _CH), jnp.float32),
            pltpu.VMEM((per_w,), jnp.float32),
            pltpu.VMEM((D, _L), jnp.float32),
            pltpu.VMEM((_L,), jnp.float32),
            pltpu.SemaphoreType.DMA,
            pltpu.SemaphoreType.DMA,
            pltpu.SemaphoreType.DMA,
        ],
    )
    def sc_a(tabT_hbm, w_hbm, b_hbm, out_hbm, x0, x1, x2, out_v, w_v, b_v,
             s0, s1, s2):
        wid = lax.axis_index("s") * _NC + lax.axis_index("c")
        col0 = v0 + wid * per_w
        pltpu.sync_copy(w_hbm, w_v)
        pltpu.sync_copy(b_hbm, b_v)
        bufs = (x0, x1, x2)
        sems = (s0, s1, s2)
        nbuf = 3

        def _chunk(k):
            return tabT_hbm.at[:, pl.ds(col0 + k * _CH, _CH)]

        def compute(x_v, k):
            base = k * _CH

            def body(i, accs):
                d = i * 2
                wv0 = w_v[d, :]
                wv1 = w_v[d + 1, :]
                return tuple(
                    accs[g]
                    + wv0 * x_v[d, pl.ds(g * _L, _L)]
                    + wv1 * x_v[d + 1, pl.ds(g * _L, _L)]
                    for g in range(n_grp)
                )

            accs = lax.fori_loop(
                0, D // 2, body,
                tuple(jnp.zeros((_L,), jnp.float32) for _ in range(n_grp))
            )
            bb = b_v[...]
            for g in range(n_grp):
                out_v[pl.ds(base + g * _L, _L)] = accs[g] + bb

        for k in range(min(nbuf - 1, n_chunks)):
            pltpu.async_copy(_chunk(k), bufs[k % nbuf], sems[k % nbuf])
        for k in range(n_chunks):
            cur = bufs[k % nbuf]
            pltpu.make_async_copy(_chunk(k), cur, sems[k % nbuf]).wait()
            if k + nbuf - 1 < n_chunks:
                pltpu.async_copy(_chunk(k + nbuf - 1),
                                 bufs[(k + nbuf - 1) % nbuf],
                                 sems[(k + nbuf - 1) % nbuf])
            compute(cur, k)
        pltpu.sync_copy(out_v, out_hbm.at[pl.ds(wid * per_w, per_w)])

    return sc_a(tableT, w_bcast, b_bcast)


def _stage_plan(V, n_stagers, bounds):
    share = V // n_stagers
    plan = []
    for w in range(n_stagers):
        lo, hi = w * share, (w + 1) * share
        segs = []
        for i, (b0, b1) in enumerate(zip(bounds[:-1], bounds[1:])):
            s, e = max(lo, b0), min(hi, b1)
            if s < e:
                segs.append((i, s - b0, s, e - s))
        plan.append(segs)
    return plan


def _sc_pool_sigmoid(t_pieces, bounds, ids_wsj, B, S):
    V = bounds[-1]
    ids_per_w = (B * S) // _NW
    rows_per_w = B // _NW
    n_acc = rows_per_w // _L
    n_stagers = 8
    plan = _stage_plan(V, n_stagers, bounds)
    bounce = 25000
    mesh = plsc.VectorSubcoreMesh(core_axis_name="c", subcore_axis_name="s")

    @functools.partial(
        pl.kernel,
        out_type=jax.ShapeDtypeStruct((B,), jnp.float32),
        mesh=mesh,
        scratch_types=[
            pltpu.VMEM((ids_per_w,), jnp.int32),
            pltpu.VMEM((ids_per_w,), jnp.float32),
            pltpu.VMEM((rows_per_w,), jnp.float32),
            pltpu.VMEM_SHARED((V,), jnp.float32),
            pltpu.SemaphoreType.DMA,
        ],
    )
    def sc_b(t0_hbm, t1_hbm, t2_hbm, ids_hbm, out_hbm,
             idx_v, vals_v, out_v, t_sh, sem):
        cid = lax.axis_index("c")
        sid = lax.axis_index("s")
        wid = sid * _NC + cid
        base = wid * ids_per_w
        srcs = (t0_hbm, t1_hbm, t2_hbm)
        for w in range(n_stagers):
            @pl.when(sid == w)
            def _(segs=plan[w]):
                for (src_i, src_off, dst_off, ln) in segs:
                    done = 0
                    while done < ln:
                        n = min(bounce, ln - done)
                        pltpu.sync_copy(
                            srcs[src_i].at[pl.ds(src_off + done, n)],
                            vals_v.at[pl.ds(0, n)])
                        pltpu.sync_copy(
                            vals_v.at[pl.ds(0, n)],
                            t_sh.at[pl.ds(dst_off + done, n)])
                        done += n

        pltpu.sync_copy(ids_hbm.at[pl.ds(base, ids_per_w)], idx_v)
        plsc.subcore_barrier()
        pltpu.async_copy(t_sh.at[idx_v], vals_v, sem).wait()
        inv = jnp.float32(1.0 / S)

        def body(s, accs):
            off = s * rows_per_w
            return tuple(
                accs[i] + vals_v[pl.ds(off + i * _L, _L)] for i in range(n_acc)
            )

        accs = lax.fori_loop(
            0, S, body, tuple(jnp.zeros((_L,), jnp.float32) for _ in range(n_acc))
        )
        for i in range(n_acc):
            y = accs[i] * inv
            out_v[pl.ds(i * _L, _L)] = 1.0 / (1.0 + jnp.exp(-y))
        pltpu.sync_copy(out_v, out_hbm.at[pl.ds(wid * rows_per_w, rows_per_w)])

    return sc_b(*t_pieces, ids_wsj)


def kernel(ids, table, W, b):
    B, S = ids.shape
    V, D = table.shape
    tableT = table.T
    WT = W.T
    w_bcast = jnp.broadcast_to(W.reshape(D, 1), (D, _L))
    b_bcast = jnp.broadcast_to(b, (_L,))
    t_sc = _sc_matvec(tableT, w_bcast, b_bcast, _V0, _V1 - _V0)
    t_head = _tc_matvec(tableT, WT, b, _V0, _BLK)
    tab_rag = lax.slice(tableT, (0, _V1), (D, V))
    t_rag = _tc_matvec(tab_rag, WT, b, V - _V1, V - _V1)
    rows_per_w = B // _NW
    ids_wsj = ids.reshape(_NW, rows_per_w, S).transpose(0, 2, 1).reshape(B * S)
    out = _sc_pool_sigmoid(
        (t_head.reshape(_V0), t_sc, t_rag.reshape(V - _V1)),
        (0, _V0, _V1, V), ids_wsj, B, S)
    return out.reshape(B, 1)


# speedup vs baseline: 1.0477x; 1.0477x over previous
import functools

import jax
import jax.numpy as jnp
from jax import lax
from jax.experimental import pallas as pl
from jax.experimental.pallas import tpu as pltpu
from jax.experimental.pallas import tpu_sc as plsc

_NC = 2
_NS = 16
_NW = _NC * _NS
_L = 16

_BLK = 32768
_V0 = 21 * _BLK
_V1 = 999424
_CH = 512


def _matvec_body(tab_ref, w_ref, b_ref, t_ref):
    x = tab_ref[...]
    w = w_ref[...]
    t_ref[...] = jnp.sum(x * w, axis=0, keepdims=True) + b_ref[0]


def _tc_matvec(tableT, WT, b, n_cols, blk):
    D = tableT.shape[0]
    grid = n_cols // blk
    return pl.pallas_call(
        _matvec_body,
        grid=(grid,),
        in_specs=[
            pl.BlockSpec((D, blk), lambda i: (0, i)),
            pl.BlockSpec((D, 1), lambda i: (0, 0)),
            pl.BlockSpec((1,), lambda i: (0,)),
        ],
        out_specs=pl.BlockSpec((1, blk), lambda i: (0, i)),
        out_shape=jax.ShapeDtypeStruct((1, n_cols), jnp.float32),
    )(tableT, WT, b)


def _sc_matvec(tableT, w_bcast, b_bcast, v0, n_cols):
    D = tableT.shape[0]
    per_w = n_cols // _NW
    n_chunks = per_w // _CH
    n_grp = _CH // _L
    mesh = plsc.VectorSubcoreMesh(core_axis_name="c", subcore_axis_name="s")

    @functools.partial(
        pl.kernel,
        out_type=jax.ShapeDtypeStruct((n_cols,), jnp.float32),
        mesh=mesh,
        scratch_types=[
            pltpu.VMEM((D, _CH), jnp.float32),
            pltpu.VMEM((D, _CH), jnp.float32),
            pltpu.VMEM((D, _CH), jnp.float32),
            pltpu.VMEM((per_w,), jnp.float32),
            pltpu.VMEM((D, _L), jnp.float32),
            pltpu.VMEM((_L,), jnp.float32),
            pltpu.SemaphoreType.DMA,
            pltpu.SemaphoreType.DMA,
            pltpu.SemaphoreType.DMA,
        ],
    )
    def sc_a(tabT_hbm, w_hbm, b_hbm, out_hbm, x0, x1, x2, out_v, w_v, b_v,
             s0, s1, s2):
        wid = lax.axis_index("s") * _NC + lax.axis_index("c")
        col0 = v0 + wid * per_w
        pltpu.sync_copy(w_hbm, w_v)
        pltpu.sync_copy(b_hbm, b_v)
        bufs = (x0, x1, x2)
        sems = (s0, s1, s2)
        nbuf = 3

        def _chunk(k):
            return tabT_hbm.at[:, pl.ds(col0 + k * _CH, _CH)]

        def compute(x_v, k):
            base = k * _CH

            def body(i, accs):
                d = i * 2
                wv0 = w_v[d, :]
                wv1 = w_v[d + 1, :]
                return tuple(
                    accs[g]
                    + wv0 * x_v[d, pl.ds(g * _L, _L)]
                    + wv1 * x_v[d + 1, pl.ds(g * _L, _L)]
                    for g in range(n_grp)
                )

            accs = lax.fori_loop(
                0, D // 2, body,
                tuple(jnp.zeros((_L,), jnp.float32) for _ in range(n_grp))
            )
            bb = b_v[...]
            for g in range(n_grp):
                out_v[pl.ds(base + g * _L, _L)] = accs[g] + bb

        for k in range(min(nbuf - 1, n_chunks)):
            pltpu.async_copy(_chunk(k), bufs[k % nbuf], sems[k % nbuf])
        for k in range(n_chunks):
            cur = bufs[k % nbuf]
            pltpu.make_async_copy(_chunk(k), cur, sems[k % nbuf]).wait()
            if k + nbuf - 1 < n_chunks:
                pltpu.async_copy(_chunk(k + nbuf - 1),
                                 bufs[(k + nbuf - 1) % nbuf],
                                 sems[(k + nbuf - 1) % nbuf])
            compute(cur, k)
        pltpu.sync_copy(out_v, out_hbm.at[pl.ds(wid * per_w, per_w)])

    return sc_a(tableT, w_bcast, b_bcast)


def _stage_plan(V, n_stagers, bounds):
    cuts = [((V * w // n_stagers) // 8) * 8 for w in range(n_stagers)] + [V]
    plan = []
    for w in range(n_stagers):
        lo, hi = cuts[w], cuts[w + 1]
        segs = []
        for i, (b0, b1) in enumerate(zip(bounds[:-1], bounds[1:])):
            s, e = max(lo, b0), min(hi, b1)
            if s < e:
                segs.append((i, s - b0, s, e - s))
        plan.append(segs)
    return plan


def _sc_pool_sigmoid(t_pieces, bounds, ids_wsj, B, S):
    V = bounds[-1]
    ids_per_w = (B * S) // _NW
    rows_per_w = B // _NW
    n_acc = rows_per_w // _L
    n_stagers = _NS
    plan = _stage_plan(V, n_stagers, bounds)
    bounce = 25000
    mesh = plsc.VectorSubcoreMesh(core_axis_name="c", subcore_axis_name="s")

    @functools.partial(
        pl.kernel,
        out_type=jax.ShapeDtypeStruct((B,), jnp.float32),
        mesh=mesh,
        scratch_types=[
            pltpu.VMEM((ids_per_w,), jnp.int32),
            pltpu.VMEM((ids_per_w,), jnp.float32),
            pltpu.VMEM((rows_per_w,), jnp.float32),
            pltpu.VMEM_SHARED((V,), jnp.float32),
            pltpu.SemaphoreType.DMA,
        ],
    )
    def sc_b(t0_hbm, t1_hbm, t2_hbm, ids_hbm, out_hbm,
             idx_v, vals_v, out_v, t_sh, sem):
        cid = lax.axis_index("c")
        sid = lax.axis_index("s")
        wid = sid * _NC + cid
        base = wid * ids_per_w
        srcs = (t0_hbm, t1_hbm, t2_hbm)
        for w in range(n_stagers):
            @pl.when(sid == w)
            def _(segs=plan[w]):
                for (src_i, src_off, dst_off, ln) in segs:
                    done = 0
                    while done < ln:
                        n = min(bounce, ln - done)
                        pltpu.sync_copy(
                            srcs[src_i].at[pl.ds(src_off + done, n)],
                            vals_v.at[pl.ds(0, n)])
                        pltpu.sync_copy(
                            vals_v.at[pl.ds(0, n)],
                            t_sh.at[pl.ds(dst_off + done, n)])
                        done += n

        pltpu.sync_copy(ids_hbm.at[pl.ds(base, ids_per_w)], idx_v)
        plsc.subcore_barrier()
        pltpu.async_copy(t_sh.at[idx_v], vals_v, sem).wait()
        inv = jnp.float32(1.0 / S)

        def body(s, accs):
            off = s * rows_per_w
            return tuple(
                accs[i] + vals_v[pl.ds(off + i * _L, _L)] for i in range(n_acc)
            )

        accs = lax.fori_loop(
            0, S, body, tuple(jnp.zeros((_L,), jnp.float32) for _ in range(n_acc))
        )
        for i in range(n_acc):
            y = accs[i] * inv
            out_v[pl.ds(i * _L, _L)] = 1.0 / (1.0 + jnp.exp(-y))
        pltpu.sync_copy(out_v, out_hbm.at[pl.ds(wid * rows_per_w, rows_per_w)])

    return sc_b(*t_pieces, ids_wsj)


def kernel(ids, table, W, b):
    B, S = ids.shape
    V, D = table.shape
    tableT = table.T
    WT = W.T
    w_bcast = jnp.broadcast_to(W.reshape(D, 1), (D, _L))
    b_bcast = jnp.broadcast_to(b, (_L,))
    t_sc = _sc_matvec(tableT, w_bcast, b_bcast, _V0, _V1 - _V0)
    t_head = _tc_matvec(tableT, WT, b, _V0, _BLK)
    tab_rag = lax.slice(tableT, (0, _V1), (D, V))
    t_rag = _tc_matvec(tab_rag, WT, b, V - _V1, V - _V1)
    rows_per_w = B // _NW
    ids_wsj = ids.reshape(_NW, rows_per_w, S).transpose(0, 2, 1).reshape(B * S)
    out = _sc_pool_sigmoid(
        (t_head.reshape(_V0), t_sc, t_rag.reshape(V - _V1)),
        (0, _V0, _V1, V), ids_wsj, B, S)
    return out.reshape(B, 1)
